# baseline (speedup 1.0000x reference)
import functools

import jax
import jax.numpy as jnp
from jax import lax
from jax.experimental import pallas as pl
from jax.experimental.pallas import tpu as pltpu
from jax.experimental.pallas import tpu_sc as plsc

_NC = 2
_NS = 16
_NW = _NC * _NS

_CHUNK = 400
_SPLITS = ((0, 128), (128, 128), (256, 128), (384, 16))


def _prep_body(x_ref, w_ref, c_ref, p_ref):
    p = jnp.dot(x_ref[...], w_ref[...], preferred_element_type=jnp.float32)
    p_ref[...] = p + c_ref[...]


def _amat_body(eblk, ea0_ref, ea1_ref, ei0_ref, ei1_ref, w0_ref, w1_ref,
               a0_ref, a1_ref, r0_ref, c0_ref, r1_ref, c1_ref):
    i = pl.program_id(0)
    t = pl.program_id(1)
    a0 = jnp.dot(ea0_ref[...], w0_ref[...], preferred_element_type=jnp.float32)
    a1 = jnp.dot(ea1_ref[...], w1_ref[...], preferred_element_type=jnp.float32)
    for tt in range(8):
        @pl.when(t == tt)
        def _(tt=tt):
            a0_ref[:, tt * 16:(tt + 1) * 16] = a0
            a1_ref[:, tt * 16:(tt + 1) * 16] = a1

    @pl.when(t == 0)
    def _():
        r0_ref[pl.ds(i * eblk, eblk)] = ei0_ref[0]
        c0_ref[pl.ds(i * eblk, eblk)] = ei0_ref[1]
        r1_ref[pl.ds(i * eblk, eblk)] = ei1_ref[0]
        c1_ref[pl.ds(i * eblk, eblk)] = ei1_ref[1]


def _unpack_body(e0p_ref, e1p_ref, e0_ref, e1_ref):
    t = pl.program_id(1)
    for tt in range(8):
        @pl.when(t == tt)
        def _(tt=tt):
            e0_ref[...] = e0p_ref[:, tt * 16:(tt + 1) * 16]
            e1_ref[...] = e1p_ref[:, tt * 16:(tt + 1) * 16]


def _node_body(nblk, n_total, x_ref, a0_ref, a1_ref, wx_ref, wa_ref, cn_ref,
               wg_ref, u_ref, bg_ref, xn_ref, un_ref, acc_ref):
    i = pl.program_id(0)
    agg = jnp.concatenate(
        [a0_ref[0] + a0_ref[1], a1_ref[0] + a1_ref[1]], axis=-1)
    xb = jnp.dot(x_ref[...], wx_ref[...], preferred_element_type=jnp.float32)
    xb = xb + jnp.dot(agg, wa_ref[...], preferred_element_type=jnp.float32)
    xb = xb + cn_ref[...]
    xn_ref[...] = xb

    @pl.when(i == 0)
    def _():
        acc_ref[...] = jnp.zeros_like(acc_ref)

    acc_ref[...] += jnp.sum(xb, axis=0, keepdims=True)

    @pl.when(i == nblk - 1)
    def _():
        mean = acc_ref[...] / jnp.float32(n_total)
        un = jnp.dot(mean, wg_ref[0:128, :],
                     preferred_element_type=jnp.float32)
        un = un + jnp.dot(u_ref[...], wg_ref[128:160, :],
                          preferred_element_type=jnp.float32)
        un_ref[...] = un + bg_ref[...]


def _sc_body(n_nodes, eq, nchunks,
             x128, a0h, a1h, row0, col0, row1, col1,
             e0p, e1p, ag0h, ag1h,
             stg, rowb, colb, colb2, rbuf, cbuf, ab, cpb,
             tabs, ags, sem, sem2):
    cid = lax.axis_index("c")
    sid = lax.axis_index("s")
    wid = cid * _NS + sid

    small = (n_nodes // _NS) // 8 * 8
    big = n_nodes - small * (_NS - 1)
    tile_lo = sid * small
    last = sid == _NS - 1

    z = jnp.zeros((16,), jnp.float32)

    def zero_agg():
        def zbody(i, c):
            cpb[i] = z
            return c

        lax.fori_loop(0, big, zbody, 0)

        @pl.when(last)
        def _():
            pltpu.sync_copy(cpb, ags.at[pl.ds(tile_lo, big)])

        @pl.when(jnp.logical_not(last))
        def _():
            pltpu.sync_copy(cpb.at[pl.ds(0, small)],
                            ags.at[pl.ds(tile_lo, small)])

    def copyout_agg(agh):
        @pl.when(last)
        def _():
            pltpu.sync_copy(ags.at[pl.ds(tile_lo, big)], cpb)
            pltpu.sync_copy(cpb, agh.at[cid, pl.ds(tile_lo, big)])

        @pl.when(jnp.logical_not(last))
        def _():
            cps = cpb.at[pl.ds(0, small)]
            pltpu.sync_copy(ags.at[pl.ds(tile_lo, small)], cps)
            pltpu.sync_copy(cps, agh.at[cid, pl.ds(tile_lo, small)])

    zero_agg()

    def stage_tables(colbase):
        def stage(half_rows):
            for h in range(4):
                lo = tile_lo + h * half_rows
                pltpu.sync_copy(x128.at[pl.ds(lo, half_rows)],
                                stg.at[pl.ds(0, half_rows)])
                for tab in (0, 1):
                    def cbody(i, c, tab=tab):
                        cpb[i] = stg[i, pl.ds(colbase + tab * 16, 16)]
                        return c
                    lax.fori_loop(0, half_rows, cbody, 0)
                    pltpu.sync_copy(cpb.at[pl.ds(0, half_rows)],
                                    tabs.at[pl.ds(tab * n_nodes + lo,
                                                  half_rows)])

        @pl.when(last)
        def _():
            stage(big // 4)

        @pl.when(jnp.logical_not(last))
        def _():
            stage(small // 4)

    for colbase, ah, rowh, colh, ep, agh in (
        (0, a0h, row0, col0, e0p, ag0h),
        (32, a1h, row1, col1, e1p, ag1h),
    ):
        stage_tables(colbase)
        plsc.subcore_barrier()
        cpr = eq // _CHUNK
        gpr = (cpr + _NW - 1) // _NW
        noff = jnp.full((16,), n_nodes, jnp.int32)

        for t in range(8):
            def chunk(g, c, ah=ah, rowh=rowh, colh=colh, ep=ep, t=t):
                c2 = g * _NW + wid

                @pl.when(c2 < cpr)
                def _(ah=ah, rowh=rowh, colh=colh, ep=ep, t=t, c2=c2):
                    q0 = c2 * _CHUNK
                    ebase = t * eq + q0
                    d1 = pltpu.async_copy(
                        rowh.at[pl.ds(ebase, _CHUNK)], rowb, sem2)
                    d2 = pltpu.async_copy(
                        colh.at[pl.ds(ebase, _CHUNK)], colb, sem2)
                    d3 = pltpu.async_copy(
                        ah.at[pl.ds(q0, _CHUNK), pl.ds(t * 16, 16)],
                        ab, sem2)
                    d1.wait()
                    d2.wait()
                    d3.wait()

                    def obody(k, cc):
                        colb2[pl.ds(k * 16, 16)] = (
                            colb[pl.ds(k * 16, 16)] + noff)
                        return cc

                    lax.fori_loop(0, _CHUNK // 16, obody, 0)
                    descs = []
                    for off, sz in _SPLITS:
                        descs.append(pltpu.async_copy(
                            tabs.at[rowb.at[pl.ds(off, sz)]],
                            rbuf.at[pl.ds(off, sz)], sem))
                        descs.append(pltpu.async_copy(
                            tabs.at[colb2.at[pl.ds(off, sz)]],
                            cbuf.at[pl.ds(off, sz)], sem))
                    for d in descs:
                        d.wait()

                    def ebody(j, cc):
                        for u in range(8):
                            m = j * 8 + u
                            v = rbuf[m] + cbuf[m] + ab[m]
                            rbuf[m] = jnp.maximum(v, 0.0)
                        return cc

                    lax.fori_loop(0, _CHUNK // 8, ebody, 0)
                    pltpu.sync_copy(
                        rbuf, ep.at[pl.ds(q0, _CHUNK), pl.ds(t * 16, 16)])
                    for off, sz in _SPLITS:
                        pltpu.sync_copy(rbuf.at[pl.ds(off, sz)],
                                        ags.at[colb.at[pl.ds(off, sz)]],
                                        add=True)

                return c

            lax.fori_loop(0, gpr, chunk, 0)

        plsc.subcore_barrier()
        copyout_agg(agh)
        if agh is ag0h:
            zero_agg()
            plsc.subcore_barrier()


def kernel(x, edge_index_0, edge_index_1, edge_attr_0, edge_attr_1, u, batch,
           We0, be0, We1, be1, Wn, bn, Wg, bg):
    n, d = x.shape
    e = edge_index_0.shape[1]
    de = edge_attr_0.shape[1]
    du = u.shape[1]
    f32 = jnp.float32

    u0 = u[0]
    wcat = jnp.concatenate(
        [We0[:d], We0[d:2 * d], We1[:d], We1[d:2 * d],
         jnp.zeros((d, d - 4 * de), f32)], axis=1)
    c0 = u0 @ We0[2 * d + de:] + be0
    c1 = u0 @ We1[2 * d + de:] + be1
    zc = jnp.zeros_like(c0)
    cc = jnp.concatenate(
        [zc, c0, zc, c1, jnp.zeros((d - 4 * de,), f32)])[None, :]
    wnx = Wn[:d]
    wna = Wn[d:d + 2 * de]
    cn = (u0 @ Wn[d + 2 * de:] + bn)[None, :]

    x128 = pl.pallas_call(
        _prep_body,
        out_shape=jax.ShapeDtypeStruct((n, d), f32),
    )(x, wcat, cc)

    blk = 2000
    eq = e // 8
    nblk_e = eq // blk
    eblk = e // nblk_e
    a0r, a1r, row0, col0, row1, col1 = pl.pallas_call(
        functools.partial(_amat_body, eblk),
        grid=(nblk_e, 8),
        in_specs=[
            pl.BlockSpec((blk, de), lambda i, t: (t * nblk_e + i, 0)),
            pl.BlockSpec((blk, de), lambda i, t: (t * nblk_e + i, 0)),
            pl.BlockSpec((2, eblk), lambda i, t: (0, i)),
            pl.BlockSpec((2, eblk), lambda i, t: (0, i)),
            pl.BlockSpec((de, de), lambda i, t: (0, 0)),
            pl.BlockSpec((de, de), lambda i, t: (0, 0)),
        ],
        out_specs=(
            pl.BlockSpec((blk, 8 * de), lambda i, t: (i, 0)),
            pl.BlockSpec((blk, 8 * de), lambda i, t: (i, 0)),
            pl.BlockSpec((e,), lambda i, t: (0,)),
            pl.BlockSpec((e,), lambda i, t: (0,)),
            pl.BlockSpec((e,), lambda i, t: (0,)),
            pl.BlockSpec((e,), lambda i, t: (0,)),
        ),
        out_shape=(
            jax.ShapeDtypeStruct((eq, 8 * de), f32),
            jax.ShapeDtypeStruct((eq, 8 * de), f32),
            jax.ShapeDtypeStruct((e,), jnp.int32),
            jax.ShapeDtypeStruct((e,), jnp.int32),
            jax.ShapeDtypeStruct((e,), jnp.int32),
            jax.ShapeDtypeStruct((e,), jnp.int32),
        ),
    )(edge_attr_0, edge_attr_1, edge_index_0, edge_index_1,
      We0[2 * d:2 * d + de], We1[2 * d:2 * d + de])

    nchunks = eq // _CHUNK

    mesh = plsc.VectorSubcoreMesh(
        core_axis_name="c", subcore_axis_name="s",
        num_cores=_NC, num_subcores=_NS)
    big = n - (n // _NS) // 8 * 8 * (_NS - 1)
    sc_fn = pl.kernel(
        functools.partial(_sc_body, n, eq, nchunks),
        out_type=(
            jax.ShapeDtypeStruct((eq, 8 * de), f32),
            jax.ShapeDtypeStruct((eq, 8 * de), f32),
            jax.ShapeDtypeStruct((_NC, n, de), f32),
            jax.ShapeDtypeStruct((_NC, n, de), f32),
        ),
        mesh=mesh,
        scratch_types=[
            pltpu.VMEM((big // 4, d), f32),
            pltpu.VMEM((_CHUNK,), jnp.int32),
            pltpu.VMEM((_CHUNK,), jnp.int32),
            pltpu.VMEM((_CHUNK,), jnp.int32),
            pltpu.VMEM((_CHUNK, de), f32),
            pltpu.VMEM((_CHUNK, de), f32),
            pltpu.VMEM((_CHUNK, de), f32),
            pltpu.VMEM((big, de), f32),
            pltpu.VMEM_SHARED((2 * n, de), f32),
            pltpu.VMEM_SHARED((n, de), f32),
            pltpu.SemaphoreType.DMA,
            pltpu.SemaphoreType.DMA,
        ],
        compiler_params=pltpu.CompilerParams(use_tc_tiling_on_sc=False),
    )
    e0p, e1p, ag0p, ag1p = sc_fn(x128, a0r, a1r, row0, col0, row1, col1)

    e0, e1 = pl.pallas_call(
        _unpack_body,
        grid=(nblk_e, 8),
        in_specs=[
            pl.BlockSpec((blk, 8 * de), lambda i, t: (i, 0)),
            pl.BlockSpec((blk, 8 * de), lambda i, t: (i, 0)),
        ],
        out_specs=(
            pl.BlockSpec((blk, de), lambda i, t: (t * nblk_e + i, 0)),
            pl.BlockSpec((blk, de), lambda i, t: (t * nblk_e + i, 0)),
        ),
        out_shape=(
            jax.ShapeDtypeStruct((e, de), f32),
            jax.ShapeDtypeStruct((e, de), f32),
        ),
    )(e0p, e1p)

    nb = 2000
    nblk_n = n // nb
    x_new, u_new = pl.pallas_call(
        functools.partial(_node_body, nblk_n, n),
        grid=(nblk_n,),
        in_specs=[
            pl.BlockSpec((nb, d), lambda i: (i, 0)),
            pl.BlockSpec((_NC, nb, de), lambda i: (0, i, 0)),
            pl.BlockSpec((_NC, nb, de), lambda i: (0, i, 0)),
            pl.BlockSpec((d, d), lambda i: (0, 0)),
            pl.BlockSpec((2 * de, d), lambda i: (0, 0)),
            pl.BlockSpec((1, d), lambda i: (0, 0)),
            pl.BlockSpec((d + du, du), lambda i: (0, 0)),
            pl.BlockSpec((1, du), lambda i: (0, 0)),
            pl.BlockSpec((1, du), lambda i: (0, 0)),
        ],
        out_specs=(
            pl.BlockSpec((nb, d), lambda i: (i, 0)),
            pl.BlockSpec((1, du), lambda i: (0, 0)),
        ),
        out_shape=(
            jax.ShapeDtypeStruct((n, d), f32),
            jax.ShapeDtypeStruct((1, du), f32),
        ),
        scratch_shapes=[pltpu.VMEM((1, d), f32)],
    )(x, ag0p, ag1p, wnx, wna, cn, Wg, u, bg[None, :])

    return (x_new, e0, e1, u_new)


# speedup vs baseline: 1.3468x; 1.3468x over previous
import functools

import jax
import jax.numpy as jnp
from jax import lax
from jax.experimental import pallas as pl
from jax.experimental.pallas import tpu as pltpu
from jax.experimental.pallas import tpu_sc as plsc

_NC = 2
_NS = 16
_NW = _NC * _NS

_CHUNK = 400
_SPLITS = ((0, 128), (128, 128), (256, 128), (384, 16))


def _prep_body(x_ref, w_ref, c_ref, p_ref):
    p = jnp.dot(x_ref[...], w_ref[...], preferred_element_type=jnp.float32)
    p_ref[...] = p + c_ref[...]


def _amat_body(eblk, ea0_ref, ea1_ref, ei0_ref, ei1_ref, w0_ref, w1_ref,
               a0_ref, a1_ref, r0_ref, c0_ref, r1_ref, c1_ref):
    a0_ref[...] = jnp.dot(ea0_ref[...], w0_ref[...],
                          preferred_element_type=jnp.float32)
    a1_ref[...] = jnp.dot(ea1_ref[...], w1_ref[...],
                          preferred_element_type=jnp.float32)
    i = pl.program_id(0)
    r0_ref[pl.ds(i * eblk, eblk)] = ei0_ref[0]
    c0_ref[pl.ds(i * eblk, eblk)] = ei0_ref[1]
    r1_ref[pl.ds(i * eblk, eblk)] = ei1_ref[0]
    c1_ref[pl.ds(i * eblk, eblk)] = ei1_ref[1]


def _node_body(nblk, n_total, x_ref, a0_ref, a1_ref, wx_ref, wa_ref, cn_ref,
               wg_ref, u_ref, bg_ref, xn_ref, un_ref, acc_ref):
    i = pl.program_id(0)
    agg = jnp.concatenate(
        [a0_ref[0] + a0_ref[1], a1_ref[0] + a1_ref[1]], axis=-1)
    xb = jnp.dot(x_ref[...], wx_ref[...], preferred_element_type=jnp.float32)
    xb = xb + jnp.dot(agg, wa_ref[...], preferred_element_type=jnp.float32)
    xb = xb + cn_ref[...]
    xn_ref[...] = xb

    @pl.when(i == 0)
    def _():
        acc_ref[...] = jnp.zeros_like(acc_ref)

    acc_ref[...] += jnp.sum(xb, axis=0, keepdims=True)

    @pl.when(i == nblk - 1)
    def _():
        mean = acc_ref[...] / jnp.float32(n_total)
        un = jnp.dot(mean, wg_ref[0:128, :],
                     preferred_element_type=jnp.float32)
        un = un + jnp.dot(u_ref[...], wg_ref[128:160, :],
                          preferred_element_type=jnp.float32)
        un_ref[...] = un + bg_ref[...]


def _sc_body(n_nodes, ew,
             x128, a0h, a1h, row0, col0, row1, col1,
             e0p, e1p, ag0h, ag1h,
             stg, rowb, colb, rbuf, cbuf, ab, ebuf, cpb,
             trs, tcs, ags, sem, sem2):
    cid = lax.axis_index("c")
    sid = lax.axis_index("s")
    wid = cid * _NS + sid

    small = (n_nodes // _NS) // 8 * 8
    big = n_nodes - small * (_NS - 1)
    tile_lo = sid * small
    last = sid == _NS - 1

    z = jnp.zeros((16,), jnp.float32)

    def zero_agg():
        def zbody(i, c):
            cpb[i] = z
            return c

        lax.fori_loop(0, big, zbody, 0)

        @pl.when(last)
        def _():
            pltpu.sync_copy(cpb, ags.at[pl.ds(tile_lo, big)])

        @pl.when(jnp.logical_not(last))
        def _():
            pltpu.sync_copy(cpb.at[pl.ds(0, small)],
                            ags.at[pl.ds(tile_lo, small)])

    def copyout_agg(agh):
        @pl.when(last)
        def _():
            pltpu.sync_copy(ags.at[pl.ds(tile_lo, big)], cpb)
            pltpu.sync_copy(cpb, agh.at[cid, pl.ds(tile_lo, big)])

        @pl.when(jnp.logical_not(last))
        def _():
            cps = cpb.at[pl.ds(0, small)]
            pltpu.sync_copy(ags.at[pl.ds(tile_lo, small)], cps)
            pltpu.sync_copy(cps, agh.at[cid, pl.ds(tile_lo, small)])

    zero_agg()

    def stage_tables(colbase):
        def stage(qrows):
            for h in range(4):
                lo = tile_lo + h * qrows
                pltpu.sync_copy(x128.at[pl.ds(lo, qrows)],
                                stg.at[pl.ds(0, qrows)])
                for tab, tsp in ((0, trs), (1, tcs)):
                    def cbody(i, c, tab=tab):
                        cpb[i] = stg[i, pl.ds(colbase + tab * 16, 16)]
                        return c
                    lax.fori_loop(0, qrows, cbody, 0)
                    pltpu.sync_copy(cpb.at[pl.ds(0, qrows)],
                                    tsp.at[pl.ds(lo, qrows)])

        @pl.when(last)
        def _():
            stage(big // 4)

        @pl.when(jnp.logical_not(last))
        def _():
            stage(small // 4)

    for colbase, ah, rowh, colh, ep, agh in (
        (0, a0h, row0, col0, e0p, ag0h),
        (32, a1h, row1, col1, e1p, ag1h),
    ):
        stage_tables(colbase)
        plsc.subcore_barrier()
        ebase0 = wid * ew
        arow0 = wid * (ew // 8)
        nchunk = ew // _CHUNK

        def start_inputs(g, ah, rowh, colh):
            p = g % 2
            ebase = ebase0 + g * _CHUNK
            arow = arow0 + g * (_CHUNK // 8)
            pltpu.async_copy(rowh.at[pl.ds(ebase, _CHUNK)], rowb.at[p], sem2)
            pltpu.async_copy(colh.at[pl.ds(ebase, _CHUNK)], colb.at[p], sem2)
            pltpu.async_copy(ah.at[pl.ds(arow, _CHUNK // 8)], ab.at[p], sem2)

        def wait_inputs(g, ah, rowh, colh):
            p = g % 2
            ebase = ebase0 + g * _CHUNK
            arow = arow0 + g * (_CHUNK // 8)
            pltpu.make_async_copy(
                rowh.at[pl.ds(ebase, _CHUNK)], rowb.at[p], sem2).wait()
            pltpu.make_async_copy(
                colh.at[pl.ds(ebase, _CHUNK)], colb.at[p], sem2).wait()
            pltpu.make_async_copy(
                ah.at[pl.ds(arow, _CHUNK // 8)], ab.at[p], sem2).wait()

        start_inputs(0, ah, rowh, colh)

        def chunk(g, c, ah=ah, rowh=rowh, colh=colh, ep=ep,
                  ebase0=ebase0, arow0=arow0, nchunk=nchunk):
            p = g % 2
            arow = arow0 + g * (_CHUNK // 8)
            wait_inputs(g, ah, rowh, colh)
            descs = []
            for off, sz in _SPLITS:
                descs.append(pltpu.async_copy(
                    trs.at[rowb.at[p, pl.ds(off, sz)]],
                    rbuf.at[pl.ds(off, sz)], sem))
                descs.append(pltpu.async_copy(
                    tcs.at[colb.at[p, pl.ds(off, sz)]],
                    cbuf.at[pl.ds(off, sz)], sem))

            @pl.when(g + 1 < nchunk)
            def _():
                start_inputs(g + 1, ah, rowh, colh)

            for d in descs:
                d.wait()

            def ebody(j, cc):
                for u in range(8):
                    m = j * 8 + u
                    v = rbuf[m] + cbuf[m] + ab[p, j, pl.ds(u * 16, 16)]
                    v = jnp.maximum(v, 0.0)
                    rbuf[m] = v
                    ebuf[j, pl.ds(u * 16, 16)] = v
                return cc

            lax.fori_loop(0, _CHUNK // 8, ebody, 0)
            pltpu.sync_copy(ebuf, ep.at[pl.ds(arow, _CHUNK // 8)])
            for off, sz in _SPLITS:
                pltpu.sync_copy(rbuf.at[pl.ds(off, sz)],
                                ags.at[colb.at[p, pl.ds(off, sz)]], add=True)
            return c

        lax.fori_loop(0, nchunk, chunk, 0)

        plsc.subcore_barrier()
        copyout_agg(agh)
        if agh is ag0h:
            zero_agg()
            plsc.subcore_barrier()


def kernel(x, edge_index_0, edge_index_1, edge_attr_0, edge_attr_1, u, batch,
           We0, be0, We1, be1, Wn, bn, Wg, bg):
    n, d = x.shape
    e = edge_index_0.shape[1]
    de = edge_attr_0.shape[1]
    du = u.shape[1]
    f32 = jnp.float32

    u0 = u[0]
    wcat = jnp.concatenate(
        [We0[:d], We0[d:2 * d], We1[:d], We1[d:2 * d],
         jnp.zeros((d, d - 4 * de), f32)], axis=1)
    c0 = u0 @ We0[2 * d + de:] + be0
    c1 = u0 @ We1[2 * d + de:] + be1
    zc = jnp.zeros_like(c0)
    cc = jnp.concatenate(
        [zc, c0, zc, c1, jnp.zeros((d - 4 * de,), f32)])[None, :]
    w8_0 = jnp.kron(jnp.eye(8, dtype=f32), We0[2 * d:2 * d + de])
    w8_1 = jnp.kron(jnp.eye(8, dtype=f32), We1[2 * d:2 * d + de])
    wnx = Wn[:d]
    wna = Wn[d:d + 2 * de]
    cn = (u0 @ Wn[d + 2 * de:] + bn)[None, :]

    x128 = pl.pallas_call(
        _prep_body,
        out_shape=jax.ShapeDtypeStruct((n, d), f32),
    )(x, wcat, cc)

    eq = e // 8
    blk = 2000
    nblk_e = eq // blk
    eblk = e // nblk_e
    ea0r = edge_attr_0.reshape(eq, 8 * de)
    ea1r = edge_attr_1.reshape(eq, 8 * de)
    a0r, a1r, row0, col0, row1, col1 = pl.pallas_call(
        functools.partial(_amat_body, eblk),
        grid=(nblk_e,),
        in_specs=[
            pl.BlockSpec((blk, 8 * de), lambda i: (i, 0)),
            pl.BlockSpec((blk, 8 * de), lambda i: (i, 0)),
            pl.BlockSpec((2, eblk), lambda i: (0, i)),
            pl.BlockSpec((2, eblk), lambda i: (0, i)),
            pl.BlockSpec((8 * de, 8 * de), lambda i: (0, 0)),
            pl.BlockSpec((8 * de, 8 * de), lambda i: (0, 0)),
        ],
        out_specs=(
            pl.BlockSpec((blk, 8 * de), lambda i: (i, 0)),
            pl.BlockSpec((blk, 8 * de), lambda i: (i, 0)),
            pl.BlockSpec((e,), lambda i: (0,)),
            pl.BlockSpec((e,), lambda i: (0,)),
            pl.BlockSpec((e,), lambda i: (0,)),
            pl.BlockSpec((e,), lambda i: (0,)),
        ),
        out_shape=(
            jax.ShapeDtypeStruct((eq, 8 * de), f32),
            jax.ShapeDtypeStruct((eq, 8 * de), f32),
            jax.ShapeDtypeStruct((e,), jnp.int32),
            jax.ShapeDtypeStruct((e,), jnp.int32),
            jax.ShapeDtypeStruct((e,), jnp.int32),
            jax.ShapeDtypeStruct((e,), jnp.int32),
        ),
    )(ea0r, ea1r, edge_index_0, edge_index_1, w8_0, w8_1)

    ew = e // _NW
    mesh = plsc.VectorSubcoreMesh(
        core_axis_name="c", subcore_axis_name="s",
        num_cores=_NC, num_subcores=_NS)
    big = n - (n // _NS) // 8 * 8 * (_NS - 1)
    sc_fn = pl.kernel(
        functools.partial(_sc_body, n, ew),
        out_type=(
            jax.ShapeDtypeStruct((eq, 8 * de), f32),
            jax.ShapeDtypeStruct((eq, 8 * de), f32),
            jax.ShapeDtypeStruct((_NC, n, de), f32),
            jax.ShapeDtypeStruct((_NC, n, de), f32),
        ),
        mesh=mesh,
        scratch_types=[
            pltpu.VMEM((big // 4, d), f32),
            pltpu.VMEM((2, _CHUNK), jnp.int32),
            pltpu.VMEM((2, _CHUNK), jnp.int32),
            pltpu.VMEM((_CHUNK, de), f32),
            pltpu.VMEM((_CHUNK, de), f32),
            pltpu.VMEM((2, _CHUNK // 8, d), f32),
            pltpu.VMEM((_CHUNK // 8, d), f32),
            pltpu.VMEM((big, de), f32),
            pltpu.VMEM_SHARED((n, de), f32),
            pltpu.VMEM_SHARED((n, de), f32),
            pltpu.VMEM_SHARED((n, de), f32),
            pltpu.SemaphoreType.DMA,
            pltpu.SemaphoreType.DMA,
        ],
        compiler_params=pltpu.CompilerParams(use_tc_tiling_on_sc=False),
    )
    e0p, e1p, ag0p, ag1p = sc_fn(x128, a0r, a1r, row0, col0, row1, col1)

    nb = 2000
    nblk_n = n // nb
    x_new, u_new = pl.pallas_call(
        functools.partial(_node_body, nblk_n, n),
        grid=(nblk_n,),
        in_specs=[
            pl.BlockSpec((nb, d), lambda i: (i, 0)),
            pl.BlockSpec((_NC, nb, de), lambda i: (0, i, 0)),
            pl.BlockSpec((_NC, nb, de), lambda i: (0, i, 0)),
            pl.BlockSpec((d, d), lambda i: (0, 0)),
            pl.BlockSpec((2 * de, d), lambda i: (0, 0)),
            pl.BlockSpec((1, d), lambda i: (0, 0)),
            pl.BlockSpec((d + du, du), lambda i: (0, 0)),
            pl.BlockSpec((1, du), lambda i: (0, 0)),
            pl.BlockSpec((1, du), lambda i: (0, 0)),
        ],
        out_specs=(
            pl.BlockSpec((nb, d), lambda i: (i, 0)),
            pl.BlockSpec((1, du), lambda i: (0, 0)),
        ),
        out_shape=(
            jax.ShapeDtypeStruct((n, d), f32),
            jax.ShapeDtypeStruct((1, du), f32),
        ),
        scratch_shapes=[pltpu.VMEM((1, d), f32)],
    )(x, ag0p, ag1p, wnx, wna, cn, Wg, u, bg[None, :])

    e0 = e0p.reshape(e, de)
    e1 = e1p.reshape(e, de)
    return (x_new, e0, e1, u_new)


# speedup vs baseline: 1.4480x; 1.0752x over previous
import functools

import jax
import jax.numpy as jnp
from jax import lax
from jax.experimental import pallas as pl
from jax.experimental.pallas import tpu as pltpu
from jax.experimental.pallas import tpu_sc as plsc

_NC = 2
_NS = 16
_NW = _NC * _NS

_CHUNK = 400
_SPLITS = ((0, 128), (128, 128), (256, 128), (384, 16))


def _prep_body(x_ref, w_ref, c_ref, p_ref):
    p = jnp.dot(x_ref[...], w_ref[...], preferred_element_type=jnp.float32)
    p_ref[...] = p + c_ref[...]


def _amat_body(eblk, ea0_ref, ea1_ref, ei0_ref, ei1_ref, w0_ref, w1_ref,
               a0_ref, a1_ref, r0_ref, c0_ref, r1_ref, c1_ref):
    a0_ref[...] = jnp.dot(ea0_ref[...], w0_ref[...],
                          preferred_element_type=jnp.float32)
    a1_ref[...] = jnp.dot(ea1_ref[...], w1_ref[...],
                          preferred_element_type=jnp.float32)
    i = pl.program_id(0)
    r0_ref[pl.ds(i * eblk, eblk)] = ei0_ref[0]
    c0_ref[pl.ds(i * eblk, eblk)] = ei0_ref[1]
    r1_ref[pl.ds(i * eblk, eblk)] = ei1_ref[0]
    c1_ref[pl.ds(i * eblk, eblk)] = ei1_ref[1]


def _node_body(nblk, n_total, x_ref, a0_ref, a1_ref, wx_ref, wa_ref, cn_ref,
               wg_ref, u_ref, bg_ref, xn_ref, un_ref, acc_ref):
    i = pl.program_id(0)
    agg = jnp.concatenate(
        [a0_ref[0] + a0_ref[1], a1_ref[0] + a1_ref[1]], axis=-1)
    xb = jnp.dot(x_ref[...], wx_ref[...], preferred_element_type=jnp.float32)
    xb = xb + jnp.dot(agg, wa_ref[...], preferred_element_type=jnp.float32)
    xb = xb + cn_ref[...]
    xn_ref[...] = xb

    @pl.when(i == 0)
    def _():
        acc_ref[...] = jnp.zeros_like(acc_ref)

    acc_ref[...] += jnp.sum(xb, axis=0, keepdims=True)

    @pl.when(i == nblk - 1)
    def _():
        mean = acc_ref[...] / jnp.float32(n_total)
        un = jnp.dot(mean, wg_ref[0:128, :],
                     preferred_element_type=jnp.float32)
        un = un + jnp.dot(u_ref[...], wg_ref[128:160, :],
                          preferred_element_type=jnp.float32)
        un_ref[...] = un + bg_ref[...]


def _sc_body(n_nodes, ew,
             x128, a0h, a1h, row0, col0, row1, col1,
             e0p, e1p, ag0h, ag1h,
             stg, rowb, colb, rbuf, cbuf, ab, ebuf, cpb,
             trs, tcs, ags, sem, sem2):
    cid = lax.axis_index("c")
    sid = lax.axis_index("s")
    wid = cid * _NS + sid

    small = (n_nodes // _NS) // 8 * 8
    big = n_nodes - small * (_NS - 1)
    tile_lo = sid * small
    last = sid == _NS - 1

    z = jnp.zeros((16,), jnp.float32)

    def zero_agg():
        def zbody(i, c):
            cpb[i] = z
            return c

        lax.fori_loop(0, big, zbody, 0)

        @pl.when(last)
        def _():
            pltpu.sync_copy(cpb, ags.at[pl.ds(tile_lo, big)])

        @pl.when(jnp.logical_not(last))
        def _():
            pltpu.sync_copy(cpb.at[pl.ds(0, small)],
                            ags.at[pl.ds(tile_lo, small)])

    def copyout_agg(agh):
        @pl.when(last)
        def _():
            pltpu.sync_copy(ags.at[pl.ds(tile_lo, big)], cpb)
            pltpu.sync_copy(cpb, agh.at[cid, pl.ds(tile_lo, big)])

        @pl.when(jnp.logical_not(last))
        def _():
            cps = cpb.at[pl.ds(0, small)]
            pltpu.sync_copy(ags.at[pl.ds(tile_lo, small)], cps)
            pltpu.sync_copy(cps, agh.at[cid, pl.ds(tile_lo, small)])

    zero_agg()

    def stage_tables(colbase):
        def stage(qrows):
            for h in range(4):
                lo = tile_lo + h * qrows
                pltpu.sync_copy(x128.at[pl.ds(lo, qrows)],
                                stg.at[pl.ds(0, qrows)])
                for tab, tsp in ((0, trs), (1, tcs)):
                    def cbody(i, c, tab=tab):
                        cpb[i] = stg[i, pl.ds(colbase + tab * 16, 16)]
                        return c
                    lax.fori_loop(0, qrows, cbody, 0)
                    pltpu.sync_copy(cpb.at[pl.ds(0, qrows)],
                                    tsp.at[pl.ds(lo, qrows)])

        @pl.when(last)
        def _():
            stage(big // 4)

        @pl.when(jnp.logical_not(last))
        def _():
            stage(small // 4)

    for colbase, ah, rowh, colh, ep, agh in (
        (0, a0h, row0, col0, e0p, ag0h),
        (32, a1h, row1, col1, e1p, ag1h),
    ):
        stage_tables(colbase)
        plsc.subcore_barrier()
        ebase0 = wid * ew
        arow0 = wid * (ew // 8)

        def chunk(g, c, ah=ah, rowh=rowh, colh=colh, ep=ep,
                  ebase0=ebase0, arow0=arow0):
            ebase = ebase0 + g * _CHUNK
            arow = arow0 + g * (_CHUNK // 8)
            d1 = pltpu.async_copy(rowh.at[pl.ds(ebase, _CHUNK)], rowb, sem2)
            d2 = pltpu.async_copy(colh.at[pl.ds(ebase, _CHUNK)], colb, sem2)
            d3 = pltpu.async_copy(ah.at[pl.ds(arow, _CHUNK // 8)], ab, sem2)
            d1.wait()
            d2.wait()
            d3.wait()
            descs = []
            for off, sz in _SPLITS:
                descs.append(pltpu.async_copy(
                    trs.at[rowb.at[pl.ds(off, sz)]],
                    rbuf.at[pl.ds(off, sz)], sem))
                descs.append(pltpu.async_copy(
                    tcs.at[colb.at[pl.ds(off, sz)]],
                    cbuf.at[pl.ds(off, sz)], sem))
            for d in descs:
                d.wait()

            def ebody(j, cc):
                for u in range(8):
                    m = j * 8 + u
                    v = rbuf[m] + cbuf[m] + ab[j, pl.ds(u * 16, 16)]
                    v = jnp.maximum(v, 0.0)
                    rbuf[m] = v
                    ebuf[j, pl.ds(u * 16, 16)] = v
                return cc

            lax.fori_loop(0, _CHUNK // 8, ebody, 0)
            pltpu.sync_copy(ebuf, ep.at[pl.ds(arow, _CHUNK // 8)])
            for off, sz in _SPLITS:
                pltpu.sync_copy(rbuf.at[pl.ds(off, sz)],
                                ags.at[colb.at[pl.ds(off, sz)]], add=True)
            return c

        lax.fori_loop(0, ew // _CHUNK, chunk, 0)

        plsc.subcore_barrier()
        copyout_agg(agh)
        if agh is ag0h:
            zero_agg()
            plsc.subcore_barrier()


def kernel(x, edge_index_0, edge_index_1, edge_attr_0, edge_attr_1, u, batch,
           We0, be0, We1, be1, Wn, bn, Wg, bg):
    n, d = x.shape
    e = edge_index_0.shape[1]
    de = edge_attr_0.shape[1]
    du = u.shape[1]
    f32 = jnp.float32

    u0 = u[0]
    wcat = jnp.concatenate(
        [We0[:d], We0[d:2 * d], We1[:d], We1[d:2 * d],
         jnp.zeros((d, d - 4 * de), f32)], axis=1)
    c0 = u0 @ We0[2 * d + de:] + be0
    c1 = u0 @ We1[2 * d + de:] + be1
    zc = jnp.zeros_like(c0)
    cc = jnp.concatenate(
        [zc, c0, zc, c1, jnp.zeros((d - 4 * de,), f32)])[None, :]
    w8_0 = jnp.kron(jnp.eye(8, dtype=f32), We0[2 * d:2 * d + de])
    w8_1 = jnp.kron(jnp.eye(8, dtype=f32), We1[2 * d:2 * d + de])
    wnx = Wn[:d]
    wna = Wn[d:d + 2 * de]
    cn = (u0 @ Wn[d + 2 * de:] + bn)[None, :]

    x128 = pl.pallas_call(
        _prep_body,
        out_shape=jax.ShapeDtypeStruct((n, d), f32),
    )(x, wcat, cc)

    eq = e // 8
    blk = 2000
    nblk_e = eq // blk
    eblk = e // nblk_e
    ea0r = edge_attr_0.reshape(eq, 8 * de)
    ea1r = edge_attr_1.reshape(eq, 8 * de)
    a0r, a1r, row0, col0, row1, col1 = pl.pallas_call(
        functools.partial(_amat_body, eblk),
        grid=(nblk_e,),
        in_specs=[
            pl.BlockSpec((blk, 8 * de), lambda i: (i, 0)),
            pl.BlockSpec((blk, 8 * de), lambda i: (i, 0)),
            pl.BlockSpec((2, eblk), lambda i: (0, i)),
            pl.BlockSpec((2, eblk), lambda i: (0, i)),
            pl.BlockSpec((8 * de, 8 * de), lambda i: (0, 0)),
            pl.BlockSpec((8 * de, 8 * de), lambda i: (0, 0)),
        ],
        out_specs=(
            pl.BlockSpec((blk, 8 * de), lambda i: (i, 0)),
            pl.BlockSpec((blk, 8 * de), lambda i: (i, 0)),
            pl.BlockSpec((e,), lambda i: (0,)),
            pl.BlockSpec((e,), lambda i: (0,)),
            pl.BlockSpec((e,), lambda i: (0,)),
            pl.BlockSpec((e,), lambda i: (0,)),
        ),
        out_shape=(
            jax.ShapeDtypeStruct((eq, 8 * de), f32),
            jax.ShapeDtypeStruct((eq, 8 * de), f32),
            jax.ShapeDtypeStruct((e,), jnp.int32),
            jax.ShapeDtypeStruct((e,), jnp.int32),
            jax.ShapeDtypeStruct((e,), jnp.int32),
            jax.ShapeDtypeStruct((e,), jnp.int32),
        ),
    )(ea0r, ea1r, edge_index_0, edge_index_1, w8_0, w8_1)

    ew = e // _NW
    mesh = plsc.VectorSubcoreMesh(
        core_axis_name="c", subcore_axis_name="s",
        num_cores=_NC, num_subcores=_NS)
    big = n - (n // _NS) // 8 * 8 * (_NS - 1)
    sc_fn = pl.kernel(
        functools.partial(_sc_body, n, ew),
        out_type=(
            jax.ShapeDtypeStruct((eq, 8 * de), f32),
            jax.ShapeDtypeStruct((eq, 8 * de), f32),
            jax.ShapeDtypeStruct((_NC, n, de), f32),
            jax.ShapeDtypeStruct((_NC, n, de), f32),
        ),
        mesh=mesh,
        scratch_types=[
            pltpu.VMEM((big // 4, d), f32),
            pltpu.VMEM((_CHUNK,), jnp.int32),
            pltpu.VMEM((_CHUNK,), jnp.int32),
            pltpu.VMEM((_CHUNK, de), f32),
            pltpu.VMEM((_CHUNK, de), f32),
            pltpu.VMEM((_CHUNK // 8, d), f32),
            pltpu.VMEM((_CHUNK // 8, d), f32),
            pltpu.VMEM((big, de), f32),
            pltpu.VMEM_SHARED((n, de), f32),
            pltpu.VMEM_SHARED((n, de), f32),
            pltpu.VMEM_SHARED((n, de), f32),
            pltpu.SemaphoreType.DMA,
            pltpu.SemaphoreType.DMA,
        ],
        compiler_params=pltpu.CompilerParams(use_tc_tiling_on_sc=False),
    )
    e0p, e1p, ag0p, ag1p = sc_fn(x128, a0r, a1r, row0, col0, row1, col1)

    nb = 2000
    nblk_n = n // nb
    x_new, u_new = pl.pallas_call(
        functools.partial(_node_body, nblk_n, n),
        grid=(nblk_n,),
        in_specs=[
            pl.BlockSpec((nb, d), lambda i: (i, 0)),
            pl.BlockSpec((_NC, nb, de), lambda i: (0, i, 0)),
            pl.BlockSpec((_NC, nb, de), lambda i: (0, i, 0)),
            pl.BlockSpec((d, d), lambda i: (0, 0)),
            pl.BlockSpec((2 * de, d), lambda i: (0, 0)),
            pl.BlockSpec((1, d), lambda i: (0, 0)),
            pl.BlockSpec((d + du, du), lambda i: (0, 0)),
            pl.BlockSpec((1, du), lambda i: (0, 0)),
            pl.BlockSpec((1, du), lambda i: (0, 0)),
        ],
        out_specs=(
            pl.BlockSpec((nb, d), lambda i: (i, 0)),
            pl.BlockSpec((1, du), lambda i: (0, 0)),
        ),
        out_shape=(
            jax.ShapeDtypeStruct((n, d), f32),
            jax.ShapeDtypeStruct((1, du), f32),
        ),
        scratch_shapes=[pltpu.VMEM((1, d), f32)],
    )(x, ag0p, ag1p, wnx, wna, cn, Wg, u, bg[None, :])

    e0 = e0p.reshape(e, de)
    e1 = e1p.reshape(e, de)
    return (x_new, e0, e1, u_new)
